# Initial kernel scaffold; baseline (speedup 1.0000x reference)
import functools

import jax
import jax.numpy as jnp
from jax import lax
from jax.experimental import pallas as pl
from jax.experimental.pallas import tpu as pltpu
from jax.experimental.pallas import tpu_sc as plsc


@functools.lru_cache(maxsize=None)
def _build_gather(n_rows, d, nc, ns):
    nw = nc * ns
    assert n_rows % nw == 0
    b_per_w = n_rows // nw
    chunk = 100
    assert b_per_w % chunk == 0
    n_chunks = b_per_w // chunk

    mesh = plsc.VectorSubcoreMesh(core_axis_name="c", subcore_axis_name="s",
                                  num_cores=nc, num_subcores=ns)

    @functools.partial(
        pl.kernel,
        out_type=jax.ShapeDtypeStruct((n_rows, d), jnp.float32),
        mesh=mesh,
        scratch_types=[
            pltpu.VMEM((n_chunks, chunk), jnp.int32),
            pltpu.VMEM((chunk, d), jnp.float32),
            pltpu.SemaphoreType.DMA,
        ],
    )
    def gather_kernel(idx_hbm, table_hbm, out_hbm, idx_v, rows_v, gsem):
        wid = lax.axis_index("s") * nc + lax.axis_index("c")
        base = wid * b_per_w
        pltpu.sync_copy(idx_hbm.at[wid], idx_v)

        def chunk_body(ci, carry):
            pltpu.async_copy(table_hbm.at[idx_v.at[ci]], rows_v, gsem).wait()
            pltpu.sync_copy(rows_v, out_hbm.at[pl.ds(base + ci * chunk, chunk)])
            return carry

        lax.fori_loop(0, n_chunks, chunk_body, 0)

    def run(ids_flat, table):
        idx3 = ids_flat.reshape(nw, n_chunks, chunk)
        return gather_kernel(idx3, table)

    return run


def kernel(input_ids, embed_weight, plus_mask, minus_mask):
    b, l = input_ids.shape
    ids_flat = input_ids.reshape(b * l).astype(jnp.int32)
    info = plsc.get_sparse_core_info()
    run = _build_gather(b * l, embed_weight.shape[1],
                        info.num_cores, info.num_subcores)
    out = run(ids_flat, embed_weight)
    return out.reshape(b, l, embed_weight.shape[1])


# speedup vs baseline: 2.0951x; 2.0951x over previous
import functools

import jax
import jax.numpy as jnp
from jax import lax
from jax.experimental import pallas as pl
from jax.experimental.pallas import tpu as pltpu
from jax.experimental.pallas import tpu_sc as plsc


@functools.lru_cache(maxsize=None)
def _build_gather(n_rows, d, nc, ns):
    nw = nc * ns
    assert n_rows % nw == 0
    b_per_w = n_rows // nw
    chunk = 128
    assert b_per_w % chunk == 0
    n_chunks = b_per_w // chunk

    mesh = plsc.VectorSubcoreMesh(core_axis_name="c", subcore_axis_name="s",
                                  num_cores=nc, num_subcores=ns)

    @functools.partial(
        pl.kernel,
        out_type=jax.ShapeDtypeStruct((n_rows, d), jnp.float32),
        mesh=mesh,
        scratch_types=[
            pltpu.VMEM((n_chunks, chunk), jnp.int32),
            pltpu.VMEM((chunk, d), jnp.float32),
            pltpu.SemaphoreType.DMA,
        ],
    )
    def gather_kernel(idx_hbm, table_hbm, out_hbm, idx_v, rows_v, gsem):
        wid = lax.axis_index("s") * nc + lax.axis_index("c")
        base = wid * b_per_w
        pltpu.sync_copy(idx_hbm.at[wid], idx_v)

        def chunk_body(ci, carry):
            pltpu.async_copy(table_hbm.at[idx_v.at[ci]], rows_v, gsem).wait()
            pltpu.sync_copy(rows_v, out_hbm.at[pl.ds(base + ci * chunk, chunk)])
            return carry

        lax.fori_loop(0, n_chunks, chunk_body, 0)

    def run(ids_flat, table):
        idx3 = ids_flat.reshape(nw, n_chunks, chunk)
        return gather_kernel(idx3, table)

    return run


def kernel(input_ids, embed_weight, plus_mask, minus_mask):
    b, l = input_ids.shape
    ids_flat = input_ids.reshape(b * l).astype(jnp.int32)
    info = plsc.get_sparse_core_info()
    run = _build_gather(b * l, embed_weight.shape[1],
                        info.num_cores, info.num_subcores)
    out = run(ids_flat, embed_weight)
    return out.reshape(b, l, embed_weight.shape[1])


# speedup vs baseline: 2.2259x; 1.0624x over previous
import functools

import jax
import jax.numpy as jnp
from jax import lax
from jax.experimental import pallas as pl
from jax.experimental.pallas import tpu as pltpu
from jax.experimental.pallas import tpu_sc as plsc


@functools.lru_cache(maxsize=None)
def _build_gather(n_rows, d, nc, ns):
    nw = nc * ns
    assert n_rows % nw == 0
    b_per_w = n_rows // nw
    chunk = 80
    nbuf = 2
    assert b_per_w % (chunk * nbuf) == 0
    n_chunks = b_per_w // chunk

    mesh = plsc.VectorSubcoreMesh(core_axis_name="c", subcore_axis_name="s",
                                  num_cores=nc, num_subcores=ns)

    @functools.partial(
        pl.kernel,
        out_type=jax.ShapeDtypeStruct((n_rows, d), jnp.float32),
        mesh=mesh,
        scratch_types=[
            pltpu.VMEM((n_chunks, chunk), jnp.int32),
        ] + [pltpu.VMEM((chunk, d), jnp.float32) for _ in range(nbuf)]
          + [pltpu.SemaphoreType.DMA for _ in range(2 * nbuf)],
    )
    def gather_kernel(idx_hbm, table_hbm, out_hbm, idx_v, *bufs):
        rows = bufs[:nbuf]
        gsem = bufs[nbuf:2 * nbuf]
        ssem = bufs[2 * nbuf:]
        wid = lax.axis_index("s") * nc + lax.axis_index("c")
        base = wid * b_per_w
        pltpu.sync_copy(idx_hbm.at[wid], idx_v)

        def gather_desc(ci, b):
            return pltpu.make_async_copy(table_hbm.at[idx_v.at[ci]],
                                         rows[b], gsem[b])

        def scatter_desc(ci, b):
            return pltpu.make_async_copy(
                rows[b], out_hbm.at[pl.ds(base + ci * chunk, chunk)], ssem[b])

        for b in range(nbuf):
            gather_desc(b, b).start()

        def outer(i, carry):
            c0 = i * nbuf
            for b in range(nbuf):
                c = c0 + b
                gather_desc(c, b).wait()
                scatter_desc(c, b).start()
                scatter_desc(c, b).wait()

                @pl.when(c + nbuf < n_chunks)
                def _():
                    gather_desc(c + nbuf, b).start()
            return carry

        lax.fori_loop(0, n_chunks // nbuf, outer, 0)

    def run(ids_flat, table):
        idx3 = ids_flat.reshape(nw, n_chunks, chunk)
        return gather_kernel(idx3, table)

    return run


def kernel(input_ids, embed_weight, plus_mask, minus_mask):
    b, l = input_ids.shape
    ids_flat = input_ids.reshape(b * l).astype(jnp.int32)
    info = plsc.get_sparse_core_info()
    run = _build_gather(b * l, embed_weight.shape[1],
                        info.num_cores, info.num_subcores)
    out = run(ids_flat, embed_weight)
    return out.reshape(b, l, embed_weight.shape[1])
